# baseline (device time: 28029 ns/iter reference)
import jax
import jax.numpy as jnp
from jax import lax
from jax.experimental import pallas as pl
from jax.experimental.pallas import tpu as pltpu


def _exchange(x16, dest2d):
    m, n = x16.shape
    dm, dn = dest2d.shape

    def body(x_ref, d_ref, xo_ref, do_ref, sx, sd, rx, rd):
        my_x = lax.axis_index("x")
        my_y = lax.axis_index("y")
        peer = (my_x, 1 - my_y)

        barrier = pltpu.get_barrier_semaphore()
        pl.semaphore_signal(
            barrier, inc=1, device_id=peer,
            device_id_type=pl.DeviceIdType.MESH,
        )
        pl.semaphore_wait(barrier, 1)

        rdma_x = pltpu.make_async_remote_copy(
            src_ref=x_ref, dst_ref=xo_ref,
            send_sem=sx, recv_sem=rx,
            device_id=peer, device_id_type=pl.DeviceIdType.MESH,
        )
        rdma_d = pltpu.make_async_remote_copy(
            src_ref=d_ref, dst_ref=do_ref,
            send_sem=sd, recv_sem=rd,
            device_id=peer, device_id_type=pl.DeviceIdType.MESH,
        )
        rdma_x.start()
        rdma_d.start()
        rdma_x.wait()
        rdma_d.wait()

    return pl.pallas_call(
        body,
        out_shape=[
            jax.ShapeDtypeStruct((m, n), x16.dtype),
            jax.ShapeDtypeStruct((dm, dn), dest2d.dtype),
        ],
        in_specs=[
            pl.BlockSpec(memory_space=pltpu.VMEM),
            pl.BlockSpec(memory_space=pltpu.VMEM),
        ],
        out_specs=[
            pl.BlockSpec(memory_space=pltpu.VMEM),
            pl.BlockSpec(memory_space=pltpu.VMEM),
        ],
        scratch_shapes=[
            pltpu.SemaphoreType.DMA,
            pltpu.SemaphoreType.DMA,
            pltpu.SemaphoreType.DMA,
            pltpu.SemaphoreType.DMA,
        ],
        compiler_params=pltpu.CompilerParams(collective_id=0),
    )(x16, dest2d)


def kernel(x, dest):
    m, n = x.shape
    x16 = x.astype(jnp.bfloat16)
    dest2d = dest.reshape(8, -1)

    peer_x16, peer_dest2d = _exchange(x16, dest2d)
    peer_dest = peer_dest2d.reshape(-1)

    my_y = lax.axis_index("y")
    first = my_y == 0
    xa = jnp.where(first, x16, peer_x16)
    xb = jnp.where(first, peer_x16, x16)
    da = jnp.where(first, dest, peer_dest)
    db = jnp.where(first, peer_dest, dest)
    full_x = jnp.concatenate([xa, xb], axis=0)
    full_d = jnp.concatenate([da, db], axis=0)

    order = jnp.argsort(full_d, stable=True)
    idx = lax.dynamic_slice_in_dim(order, my_y * m, m)
    return jnp.take(full_x, idx, axis=0)


# device time: 18774 ns/iter; 1.4930x vs baseline; 1.4930x over previous
import jax
import jax.numpy as jnp
from jax import lax
from jax.experimental import pallas as pl
from jax.experimental.pallas import tpu as pltpu

N_CHUNKS = 4


def _excl_cumsum_and_total(mask_i32):
    m = mask_i32.shape[1]
    lane = lax.broadcasted_iota(jnp.int32, (1, m), 1)
    inc = mask_i32
    k = 1
    while k < m:
        shifted = jnp.where(lane >= k, jnp.roll(inc, k, axis=1), 0)
        inc = inc + shifted
        k *= 2
    total = inc[:, m - 1 : m]
    return inc - mask_i32, total


def kernel(x, dest):
    m, n = x.shape
    c = m // N_CHUNKS
    dest2d = dest.reshape(1, m)

    def body(x_ref, d_ref, out_ref, xmine_ref, xrecv_ref, drecv_ref,
             dsend_sem, drecv_sem, xsend_sems, xrecv_sems):
        my_x = lax.axis_index("x")
        my_y = lax.axis_index("y")
        peer = (my_x, 1 - my_y)

        barrier = pltpu.get_barrier_semaphore()
        pl.semaphore_signal(
            barrier, inc=1, device_id=peer,
            device_id_type=pl.DeviceIdType.MESH,
        )
        pl.semaphore_wait(barrier, 1)

        xmine_ref[...] = x_ref[...].astype(jnp.bfloat16)

        rdma_d = pltpu.make_async_remote_copy(
            src_ref=d_ref, dst_ref=drecv_ref,
            send_sem=dsend_sem, recv_sem=drecv_sem,
            device_id=peer, device_id_type=pl.DeviceIdType.MESH,
        )
        rdma_d.start()
        rdma_x = []
        for j in range(N_CHUNKS):
            r = pltpu.make_async_remote_copy(
                src_ref=xmine_ref.at[pl.ds(j * c, c), :],
                dst_ref=xrecv_ref.at[pl.ds(j * c, c), :],
                send_sem=xsend_sems.at[j], recv_sem=xrecv_sems.at[j],
                device_id=peer, device_id_type=pl.DeviceIdType.MESH,
            )
            r.start()
            rdma_x.append(r)

        d_mine = d_ref[...]
        m_mine = (d_mine == my_y).astype(jnp.int32)
        e_mine, tot_keep = _excl_cumsum_and_total(m_mine)
        my_off = (m - tot_keep) * my_y
        pos_mine = jnp.where(m_mine == 1, my_off + e_mine, 2 * m)

        row = lax.broadcasted_iota(jnp.int32, (m, m), 0)
        p_mine = (row == pos_mine).astype(jnp.bfloat16)
        acc = jnp.dot(p_mine, xmine_ref[...],
                      preferred_element_type=jnp.float32)

        rdma_d.wait()
        d_peer = drecv_ref[...]
        m_peer = (d_peer == my_y).astype(jnp.int32)
        e_peer, _ = _excl_cumsum_and_total(m_peer)
        peer_off = tot_keep * (1 - my_y)
        pos_peer = jnp.where(m_peer == 1, peer_off + e_peer, 2 * m)
        p_peer = (row == pos_peer).astype(jnp.bfloat16)

        for j in range(N_CHUNKS):
            rdma_x[j].wait()
            sl = pl.ds(j * c, c)
            acc = acc + jnp.dot(
                p_peer[:, j * c : (j + 1) * c], xrecv_ref[sl, :],
                preferred_element_type=jnp.float32,
            )

        out_ref[...] = acc.astype(jnp.bfloat16)

    return pl.pallas_call(
        body,
        out_shape=jax.ShapeDtypeStruct((m, n), jnp.bfloat16),
        in_specs=[
            pl.BlockSpec(memory_space=pltpu.VMEM),
            pl.BlockSpec(memory_space=pltpu.VMEM),
        ],
        out_specs=pl.BlockSpec(memory_space=pltpu.VMEM),
        scratch_shapes=[
            pltpu.VMEM((m, n), jnp.bfloat16),
            pltpu.VMEM((m, n), jnp.bfloat16),
            pltpu.VMEM((1, m), jnp.int32),
            pltpu.SemaphoreType.DMA,
            pltpu.SemaphoreType.DMA,
            pltpu.SemaphoreType.DMA((N_CHUNKS,)),
            pltpu.SemaphoreType.DMA((N_CHUNKS,)),
        ],
        compiler_params=pltpu.CompilerParams(collective_id=0),
    )(x, dest2d)


# device time: 18163 ns/iter; 1.5432x vs baseline; 1.0336x over previous
import jax
import jax.numpy as jnp
from jax import lax
from jax.experimental import pallas as pl
from jax.experimental.pallas import tpu as pltpu

C = 128


def kernel(x, dest):
    m, n = x.shape
    k_max = m // C
    dest2d = dest.reshape(1, m)
    c0_arr = jnp.sum(jnp.where(dest == 0, 1, 0)).astype(jnp.int32).reshape(1)

    def body(x_ref, d_ref, c0_ref, out_ref, sbuf_ref, rbuf_ref,
             send_sems, recv_sems):
        my_x = lax.axis_index("x")
        my_y = lax.axis_index("y")
        peer = (my_x, 1 - my_y)

        rbuf_ref[...] = jnp.zeros((m, n), jnp.bfloat16)

        barrier = pltpu.get_barrier_semaphore()
        pl.semaphore_signal(
            barrier, inc=1, device_id=peer,
            device_id_type=pl.DeviceIdType.MESH,
        )
        pl.semaphore_wait(barrier, 1)

        d = d_ref[...]
        mask0 = (d == 0).astype(jnp.int32)
        lane = lax.broadcasted_iota(jnp.int32, (1, m), 1)
        inc = mask0
        k = 1
        while k < m:
            inc = inc + jnp.where(lane >= k, jnp.roll(inc, k, axis=1), 0)
            k *= 2
        e0 = inc - mask0
        c0_vec = inc[:, m - 1 : m]
        e1 = lane - e0

        send_sel = d == (1 - my_y)
        pos_send = jnp.where(send_sel, jnp.where(my_y == 0, e1, e0), 2 * m)
        keep_sel = d == my_y
        pos_keep = jnp.where(
            keep_sel, jnp.where(my_y == 0, e0, c0_vec + e1), 2 * m
        )

        row = lax.broadcasted_iota(jnp.int32, (m, m), 0)
        xb = x_ref[...].astype(jnp.bfloat16)
        p_send = (row == pos_send).astype(jnp.bfloat16)
        sbuf_ref[...] = jnp.dot(
            p_send, xb, preferred_element_type=jnp.float32
        ).astype(jnp.bfloat16)

        c0 = c0_ref[0]
        a = jnp.where(my_y == 0, c0, m - c0)
        s = m - a
        length = ((s + 7) // 8) * 8
        k_send = (length + C - 1) // C
        peer_off = (1 - my_y) * c0

        rdmas = []
        for j in range(k_max):
            rel = jnp.minimum(j * C, length - C)
            r = pltpu.make_async_remote_copy(
                src_ref=sbuf_ref.at[pl.ds(rel, C), :],
                dst_ref=rbuf_ref.at[pl.ds(rel, C), :],
                send_sem=send_sems.at[j], recv_sem=recv_sems.at[j],
                device_id=peer, device_id_type=pl.DeviceIdType.MESH,
            )
            rdmas.append(r)

            @pl.when(j < k_send)
            def _():
                r.start()

        p_keep = (row == pos_keep).astype(jnp.bfloat16)
        acc = jnp.dot(p_keep, xb, preferred_element_type=jnp.float32)

        col = lax.broadcasted_iota(jnp.int32, (m, m), 1)
        band = (row == col + peer_off).astype(jnp.bfloat16)

        for j in range(k_max):

            @pl.when(j < k_send)
            def _():
                rdmas[j].wait_recv()

            acc = acc + jnp.dot(
                band[:, j * C : (j + 1) * C],
                rbuf_ref[pl.ds(j * C, C), :],
                preferred_element_type=jnp.float32,
            )

        out_ref[...] = acc.astype(jnp.bfloat16)

        for j in range(k_max):

            @pl.when(j < k_send)
            def _():
                rdmas[j].wait_send()

    return pl.pallas_call(
        body,
        out_shape=jax.ShapeDtypeStruct((m, n), jnp.bfloat16),
        in_specs=[
            pl.BlockSpec(memory_space=pltpu.VMEM),
            pl.BlockSpec(memory_space=pltpu.VMEM),
            pl.BlockSpec(memory_space=pltpu.SMEM),
        ],
        out_specs=pl.BlockSpec(memory_space=pltpu.VMEM),
        scratch_shapes=[
            pltpu.VMEM((m, n), jnp.bfloat16),
            pltpu.VMEM((m, n), jnp.bfloat16),
            pltpu.SemaphoreType.DMA((m // C,)),
            pltpu.SemaphoreType.DMA((m // C,)),
        ],
        compiler_params=pltpu.CompilerParams(collective_id=0),
    )(x, dest2d, c0_arr)
